# baseline (speedup 1.0000x reference)
import functools

import jax
import jax.numpy as jnp
from jax import lax
from jax.experimental import pallas as pl
from jax.experimental.pallas import tpu as pltpu
from jax.experimental.pallas import tpu_sc as plsc

N = 10000
D = 128
G = 256

NC, NS, L = 2, 16, 16
NW = NC * NS

N_PAD = 10240
BLK = 1024
NBLK = N_PAD // BLK
ROWS_PER_TILE = N_PAD // NS

E = 320000
ECHUNK = 128
CH = 80
MCH = 40
NPH = CH // MCH
E_PAD = NW * CH * ECHUNK

GCHUNK = 64
EMB_CH = 5
EMB_PER_W = GCHUNK * EMB_CH


def _sc_gather_deg_body(table_hbm, idx_hbm, dst3_hbm, ea3_hbm,
                        h_hbm, degp_hbm,
                        dacc_sh, zbuf, idx2d, erows_v, didx2d, ea_v, sem):
    c = lax.axis_index("c")
    s = lax.axis_index("s")
    wid = s * NC + c

    def zb(r, _):
        zbuf[pl.ds(r * L, L)] = jnp.zeros((L,), jnp.float32)
        return 0
    lax.fori_loop(0, ROWS_PER_TILE // L, zb, 0)
    pltpu.sync_copy(zbuf, dacc_sh.at[pl.ds(s * ROWS_PER_TILE, ROWS_PER_TILE)])
    plsc.subcore_barrier()

    pltpu.sync_copy(idx_hbm.at[wid], idx2d)
    for j in range(EMB_CH):
        off = wid * EMB_PER_W + j * GCHUNK
        pltpu.async_copy(table_hbm.at[idx2d.at[j]], erows_v, sem).wait()
        pltpu.sync_copy(erows_v, h_hbm.at[pl.ds(off, GCHUNK)])

    for phase in range(CH // MCH):
        pltpu.sync_copy(dst3_hbm.at[wid * NPH + phase], didx2d)
        pltpu.sync_copy(ea3_hbm.at[wid * NPH + phase], ea_v)

        def ch_body(ch, _):
            pltpu.sync_copy(ea_v.at[pl.ds(ch * ECHUNK, ECHUNK)],
                            dacc_sh.at[didx2d.at[ch]], add=True)
            return 0
        lax.fori_loop(0, MCH, ch_body, 0)
    plsc.subcore_barrier()

    sl = pl.ds(s * ROWS_PER_TILE, ROWS_PER_TILE)
    pltpu.sync_copy(dacc_sh.at[sl], zbuf)
    pltpu.sync_copy(zbuf, degp_hbm.at[c, sl])


def _sc_gather_deg(table, idx3, dst3, ea3):
    mesh = plsc.VectorSubcoreMesh(core_axis_name="c", subcore_axis_name="s")
    k = functools.partial(
        pl.kernel,
        mesh=mesh,
        out_type=[
            jax.ShapeDtypeStruct((N_PAD, D), jnp.float32),
            jax.ShapeDtypeStruct((NC, N_PAD), jnp.float32),
        ],
        scratch_types=[
            pltpu.VMEM_SHARED((N_PAD,), jnp.float32),
            pltpu.VMEM((ROWS_PER_TILE,), jnp.float32),
            pltpu.VMEM((EMB_CH, GCHUNK), jnp.int32),
            pltpu.VMEM((GCHUNK, D), jnp.float32),
            pltpu.VMEM((MCH, ECHUNK), jnp.int32),
            pltpu.VMEM((MCH * ECHUNK,), jnp.float32),
            pltpu.SemaphoreType.DMA,
        ],
    )(_sc_gather_deg_body)
    return k(table, idx3, dst3, ea3)


def _sc_mp_body(hl2_hbm, src3_hbm, dst3_hbm, ea3_hbm, out_hbm,
                acc_sh, sidx2d, didx2d, ea_v, rows_a, rows_b,
                sem_ga, sem_gb, sem_sa, sem_sb):
    c = lax.axis_index("c")
    s = lax.axis_index("s")
    wid = s * NC + c

    def zr(r, _):
        for j in range(D // L):
            rows_a[r, pl.ds(j * L, L)] = jnp.zeros((L,), jnp.float32)
        return 0
    lax.fori_loop(0, ECHUNK, zr, 0)
    for k in range(ROWS_PER_TILE // ECHUNK):
        sl = pl.ds(s * ROWS_PER_TILE + k * ECHUNK, ECHUNK)
        pltpu.sync_copy(rows_a, acc_sh.at[sl])
    plsc.subcore_barrier()

    rows = (rows_a, rows_b)
    sem_g = (sem_ga, sem_gb)
    sem_s = (sem_sa, sem_sb)

    for phase in range(CH // MCH):
        pltpu.sync_copy(src3_hbm.at[wid * NPH + phase], sidx2d)
        pltpu.sync_copy(dst3_hbm.at[wid * NPH + phase], didx2d)
        pltpu.sync_copy(ea3_hbm.at[wid * NPH + phase], ea_v)
        pltpu.async_copy(hl2_hbm.at[sidx2d.at[0]], rows_a, sem_ga)

        def pair_body(k2, _):
            for u in (0, 1):
                ch = 2 * k2 + u
                rb, rnb = rows[u], rows[1 - u]
                sgb, sgn = sem_g[u], sem_g[1 - u]
                ssb, ssn = sem_s[u], sem_s[1 - u]

                @pl.when(ch > 0)
                def _():
                    pltpu.make_async_copy(
                        rnb, acc_sh.at[didx2d.at[ch - 1]], ssn).wait()

                @pl.when(ch < MCH - 1)
                def _():
                    pltpu.async_copy(hl2_hbm.at[sidx2d.at[ch + 1]], rnb, sgn)

                pltpu.make_async_copy(hl2_hbm.at[sidx2d.at[ch]], rb, sgb).wait()

                @plsc.parallel_loop(0, ECHUNK // L, unroll=2)
                def _(g):
                    ea16 = ea_v[pl.ds(ch * ECHUNK + g * L, L)]
                    for e in range(L):
                        eav = ea16.at[jnp.full((L,), e, jnp.int32)].get(
                            mode="promise_in_bounds")
                        r = g * L + e
                        for j in range(D // L):
                            sl = pl.ds(j * L, L)
                            rb[r, sl] = rb[r, sl] * eav
                pltpu.async_copy(rb, acc_sh.at[didx2d.at[ch]], ssb, add=True)
            return 0
        lax.fori_loop(0, MCH // 2, pair_body, 0)
        pltpu.make_async_copy(
            rows_b, acc_sh.at[didx2d.at[MCH - 1]], sem_sb).wait()
    plsc.subcore_barrier()

    for k in range(ROWS_PER_TILE // ECHUNK):
        sl = pl.ds(s * ROWS_PER_TILE + k * ECHUNK, ECHUNK)
        pltpu.sync_copy(acc_sh.at[sl], rows_a)
        pltpu.sync_copy(rows_a, out_hbm.at[c, sl])


def _sc_mp(hl2, src3, dst3, ea3):
    mesh = plsc.VectorSubcoreMesh(core_axis_name="c", subcore_axis_name="s")
    k = functools.partial(
        pl.kernel,
        mesh=mesh,
        out_type=jax.ShapeDtypeStruct((NC, N_PAD, D), jnp.float32),
        scratch_types=[
            pltpu.VMEM_SHARED((N_PAD, D), jnp.float32),
            pltpu.VMEM((MCH, ECHUNK), jnp.int32),
            pltpu.VMEM((MCH, ECHUNK), jnp.int32),
            pltpu.VMEM((MCH * ECHUNK,), jnp.float32),
            pltpu.VMEM((ECHUNK, D), jnp.float32),
            pltpu.VMEM((ECHUNK, D), jnp.float32),
            pltpu.SemaphoreType.DMA,
            pltpu.SemaphoreType.DMA,
            pltpu.SemaphoreType.DMA,
            pltpu.SemaphoreType.DMA,
        ],
    )(_sc_mp_body)
    return k(hl2, src3, dst3, ea3)


def _tc_stats_mm_body(h_ref, degp_ref, g_ref, be_ref, w_ref,
                      hl2_ref, dis_ref, stats_scr, dis_scr):
    i = pl.program_id(0)
    ib = i % NBLK

    @pl.when(i < NBLK)
    def _():
        deg = degp_ref[0] + degp_ref[1]
        dis = jnp.where(deg > 0, lax.rsqrt(jnp.maximum(deg, 1e-12)), 0.0)
        dis_scr[pl.ds(ib * BLK, BLK), :] = dis
        dis_ref[...] = dis
        rows = lax.broadcasted_iota(jnp.int32, (BLK, 1), 0) + ib * BLK
        m = (rows < N).astype(jnp.float32)
        hm = h_ref[...] * m

        @pl.when(i == 0)
        def _():
            stats_scr[...] = jnp.zeros((8, D), jnp.float32)

        stats_scr[0:1, :] += jnp.sum(hm, axis=0, keepdims=True)
        stats_scr[1:2, :] += jnp.sum(hm * h_ref[...], axis=0, keepdims=True)

    @pl.when(i >= NBLK)
    def _():
        mean = stats_scr[0:1, :] * (1.0 / N)
        var = stats_scr[1:2, :] * (1.0 / N) - mean * mean
        alpha = g_ref[...] * lax.rsqrt(var + 1e-5)
        beta = be_ref[...] - mean * alpha
        hb = h_ref[...] * alpha + beta
        hl = lax.dot_general(hb, w_ref[...], (((1,), (1,)), ((), ())),
                             preferred_element_type=jnp.float32)
        hl2_ref[...] = hl * dis_scr[pl.ds(ib * BLK, BLK), :]


def _tc_stats_mm(h, degp3, g2, be2, W):
    return pl.pallas_call(
        _tc_stats_mm_body,
        grid=(2 * NBLK,),
        in_specs=[
            pl.BlockSpec((BLK, D), lambda i: (i % NBLK, 0)),
            pl.BlockSpec((NC, BLK, 1), lambda i: (0, i % NBLK, 0)),
            pl.BlockSpec((1, D), lambda i: (0, 0)),
            pl.BlockSpec((1, D), lambda i: (0, 0)),
            pl.BlockSpec((D, D), lambda i: (0, 0)),
        ],
        out_specs=[
            pl.BlockSpec((BLK, D), lambda i: (i % NBLK, 0)),
            pl.BlockSpec((BLK, 1), lambda i: (i % NBLK, 0)),
        ],
        out_shape=[
            jax.ShapeDtypeStruct((N_PAD, D), jnp.float32),
            jax.ShapeDtypeStruct((N_PAD, 1), jnp.float32),
        ],
        scratch_shapes=[
            pltpu.VMEM((8, D), jnp.float32),
            pltpu.VMEM((N_PAD, 1), jnp.float32),
        ],
    )(h, degp3, g2, be2, W)


def _tc_ep_mm_body(p_ref, dis_ref, b_ref, g_ref, be_ref, w_ref,
                   hl2_ref, h1_scr, stats_scr):
    i = pl.program_id(0)
    ib = i % NBLK

    @pl.when(i < NBLK)
    def _():
        acc = p_ref[0] + p_ref[1]
        h = jax.nn.relu(acc * dis_ref[...] + b_ref[...])
        h1_scr[pl.ds(ib * BLK, BLK), :] = h
        rows = lax.broadcasted_iota(jnp.int32, (BLK, 1), 0) + ib * BLK
        m = (rows < N).astype(jnp.float32)
        hm = h * m

        @pl.when(i == 0)
        def _():
            stats_scr[...] = jnp.zeros((8, D), jnp.float32)

        stats_scr[0:1, :] += jnp.sum(hm, axis=0, keepdims=True)
        stats_scr[1:2, :] += jnp.sum(hm * h, axis=0, keepdims=True)

    @pl.when(i >= NBLK)
    def _():
        mean = stats_scr[0:1, :] * (1.0 / N)
        var = stats_scr[1:2, :] * (1.0 / N) - mean * mean
        alpha = g_ref[...] * lax.rsqrt(var + 1e-5)
        beta = be_ref[...] - mean * alpha
        hb = h1_scr[pl.ds(ib * BLK, BLK), :] * alpha + beta
        hl = lax.dot_general(hb, w_ref[...], (((1,), (1,)), ((), ())),
                             preferred_element_type=jnp.float32)
        hl2_ref[...] = hl * dis_ref[...]


def _tc_ep_mm(P, dis, b2, g2, be2, W):
    return pl.pallas_call(
        _tc_ep_mm_body,
        grid=(2 * NBLK,),
        in_specs=[
            pl.BlockSpec((NC, BLK, D), lambda i: (0, i % NBLK, 0)),
            pl.BlockSpec((BLK, 1), lambda i: (i % NBLK, 0)),
            pl.BlockSpec((1, D), lambda i: (0, 0)),
            pl.BlockSpec((1, D), lambda i: (0, 0)),
            pl.BlockSpec((1, D), lambda i: (0, 0)),
            pl.BlockSpec((D, D), lambda i: (0, 0)),
        ],
        out_specs=pl.BlockSpec((BLK, D), lambda i: (i % NBLK, 0)),
        out_shape=jax.ShapeDtypeStruct((N_PAD, D), jnp.float32),
        scratch_shapes=[
            pltpu.VMEM((N_PAD, D), jnp.float32),
            pltpu.VMEM((8, D), jnp.float32),
        ],
    )(P, dis, b2, g2, be2, W)


def _tc_ep_pool_body(p_ref, dis_ref, b_ref, tf_ref, batch_ref,
                     out_ref, pool_acc, s_acc):
    i = pl.program_id(0)
    acc = p_ref[0] + p_ref[1]
    h2 = jax.nn.relu(acc * dis_ref[...] + b_ref[...])
    ex = jnp.exp(tf_ref[...])
    ids = lax.broadcasted_iota(jnp.int32, (G, 1), 0)
    onehot = (batch_ref[0] == ids).astype(jnp.float32)

    @pl.when(i == 0)
    def _():
        pool_acc[...] = jnp.zeros((G, D), jnp.float32)
        s_acc[...] = jnp.zeros((G, 1), jnp.float32)

    hw = h2 * ex
    pool_acc[...] += lax.dot_general(onehot, hw, (((1,), (0,)), ((), ())),
                                     preferred_element_type=jnp.float32)
    s_acc[...] += lax.dot_general(onehot, ex, (((1,), (0,)), ((), ())),
                                  preferred_element_type=jnp.float32)

    @pl.when(i == NBLK - 1)
    def _():
        out_ref[...] = pool_acc[...] / (s_acc[...] + 1e-16)


def _tc_ep_pool(P, dis, b2, tf_col, batch3):
    return pl.pallas_call(
        _tc_ep_pool_body,
        grid=(NBLK,),
        in_specs=[
            pl.BlockSpec((NC, BLK, D), lambda i: (0, i, 0)),
            pl.BlockSpec((BLK, 1), lambda i: (i, 0)),
            pl.BlockSpec((1, D), lambda i: (0, 0)),
            pl.BlockSpec((BLK, 1), lambda i: (i, 0)),
            pl.BlockSpec((1, 1, BLK), lambda i: (i, 0, 0)),
        ],
        out_specs=pl.BlockSpec((G, D), lambda i: (0, 0)),
        out_shape=jax.ShapeDtypeStruct((G, D), jnp.float32),
        scratch_shapes=[
            pltpu.VMEM((G, D), jnp.float32),
            pltpu.VMEM((G, 1), jnp.float32),
        ],
    )(P, dis, b2, tf_col, batch3)


def kernel(x, edge_index, batch, edge_attr, emb_table,
           bn_gamma0, bn_beta0, W0, b0,
           bn_gamma1, bn_beta1, W1, b1):
    x_idx = x[:, 0].astype(jnp.int32)
    tfidf = x[:, 1]
    idx_pad = jnp.concatenate(
        [x_idx, jnp.arange(N_PAD - N, dtype=jnp.int32)])
    idx3 = idx_pad.reshape(NW, EMB_CH, GCHUNK)

    src = edge_index[0].astype(jnp.int32)
    dst = edge_index[1].astype(jnp.int32)
    epad = E_PAD - E
    fill = jnp.arange(epad, dtype=jnp.int32) % N
    src3 = jnp.concatenate([src, fill]).reshape(NW * NPH, MCH, ECHUNK)
    dst3 = jnp.concatenate([dst, fill]).reshape(NW * NPH, MCH, ECHUNK)
    ea3 = jnp.concatenate(
        [edge_attr, jnp.zeros((epad,), jnp.float32)]).reshape(
            NW * NPH, MCH * ECHUNK)

    h0, degp = _sc_gather_deg(emb_table, idx3, dst3, ea3)
    degp3 = degp.reshape(NC, N_PAD, 1)
    hl2_0, dis = _tc_stats_mm(h0, degp3, bn_gamma0.reshape(1, D),
                              bn_beta0.reshape(1, D), W0)
    P0 = _sc_mp(hl2_0, src3, dst3, ea3)
    hl2_1 = _tc_ep_mm(P0, dis, b0.reshape(1, D), bn_gamma1.reshape(1, D),
                      bn_beta1.reshape(1, D), W1)
    P1 = _sc_mp(hl2_1, src3, dst3, ea3)

    tf_col = jnp.concatenate([tfidf, jnp.zeros((N_PAD - N,), jnp.float32)])
    tf_col = tf_col.reshape(N_PAD, 1)
    batch3 = jnp.concatenate([batch.astype(jnp.int32),
                              jnp.full((N_PAD - N,), G, jnp.int32)])
    batch3 = batch3.reshape(NBLK, 1, BLK)

    return _tc_ep_pool(P1, dis, b1.reshape(1, D), tf_col, batch3)


# speedup vs baseline: 1.0408x; 1.0408x over previous
import functools

import jax
import jax.numpy as jnp
from jax import lax
from jax.experimental import pallas as pl
from jax.experimental.pallas import tpu as pltpu
from jax.experimental.pallas import tpu_sc as plsc

N = 10000
D = 128
G = 256

NC, NS, L = 2, 16, 16
NW = NC * NS

N_PAD = 10240
BLK = 1024
NBLK = N_PAD // BLK
ROWS_PER_TILE = N_PAD // NS

E = 320000
ECHUNK = 128
CH = 80
MCH = 40
NPH = CH // MCH
E_PAD = NW * CH * ECHUNK

GCHUNK = 64
EMB_CH = 5
EMB_PER_W = GCHUNK * EMB_CH


def _sc_gather_deg_body(table_hbm, idx_hbm, dst3_hbm, ea3_hbm,
                        h_hbm, degp_hbm,
                        dacc_sh, zbuf, idx2d, erows_v, erows_w, didx2d, ea_v,
                        sem, sem2):
    c = lax.axis_index("c")
    s = lax.axis_index("s")
    wid = s * NC + c

    def zb(r, _):
        zbuf[pl.ds(r * L, L)] = jnp.zeros((L,), jnp.float32)
        return 0
    lax.fori_loop(0, ROWS_PER_TILE // L, zb, 0)
    pltpu.sync_copy(zbuf, dacc_sh.at[pl.ds(s * ROWS_PER_TILE, ROWS_PER_TILE)])
    plsc.subcore_barrier()

    pltpu.sync_copy(idx_hbm.at[wid], idx2d)
    ebufs = (erows_v, erows_w)
    pltpu.async_copy(table_hbm.at[idx2d.at[0]], erows_v, sem)
    for j in range(EMB_CH):
        b = ebufs[j % 2]
        off = wid * EMB_PER_W + j * GCHUNK
        if j >= 2:
            poff = wid * EMB_PER_W + (j - 2) * GCHUNK
            pltpu.make_async_copy(b, h_hbm.at[pl.ds(poff, GCHUNK)],
                                  sem2).wait()
        pltpu.make_async_copy(table_hbm.at[idx2d.at[j]], b, sem).wait()
        if j + 1 < EMB_CH:
            pltpu.async_copy(table_hbm.at[idx2d.at[j + 1]],
                             ebufs[(j + 1) % 2], sem)
        pltpu.async_copy(b, h_hbm.at[pl.ds(off, GCHUNK)], sem2)
    for j in range(EMB_CH - 2, EMB_CH):
        b = ebufs[j % 2]
        off = wid * EMB_PER_W + j * GCHUNK
        pltpu.make_async_copy(b, h_hbm.at[pl.ds(off, GCHUNK)], sem2).wait()

    for phase in range(CH // MCH):
        pltpu.sync_copy(dst3_hbm.at[wid * NPH + phase], didx2d)
        pltpu.sync_copy(ea3_hbm.at[wid * NPH + phase], ea_v)

        def wave_body(w, _):
            for u in range(8):
                ch = w * 8 + u
                pltpu.async_copy(ea_v.at[pl.ds(ch * ECHUNK, ECHUNK)],
                                 dacc_sh.at[didx2d.at[ch]], sem, add=True)
            for u in range(8):
                ch = w * 8 + u
                pltpu.make_async_copy(
                    ea_v.at[pl.ds(ch * ECHUNK, ECHUNK)],
                    dacc_sh.at[didx2d.at[ch]], sem).wait()
            return 0
        lax.fori_loop(0, MCH // 8, wave_body, 0)
    plsc.subcore_barrier()

    sl = pl.ds(s * ROWS_PER_TILE, ROWS_PER_TILE)
    pltpu.sync_copy(dacc_sh.at[sl], zbuf)
    pltpu.sync_copy(zbuf, degp_hbm.at[c, sl])


def _sc_gather_deg(table, idx3, dst3, ea3):
    mesh = plsc.VectorSubcoreMesh(core_axis_name="c", subcore_axis_name="s")
    k = functools.partial(
        pl.kernel,
        mesh=mesh,
        out_type=[
            jax.ShapeDtypeStruct((N_PAD, D), jnp.float32),
            jax.ShapeDtypeStruct((NC, N_PAD), jnp.float32),
        ],
        scratch_types=[
            pltpu.VMEM_SHARED((N_PAD,), jnp.float32),
            pltpu.VMEM((ROWS_PER_TILE,), jnp.float32),
            pltpu.VMEM((EMB_CH, GCHUNK), jnp.int32),
            pltpu.VMEM((GCHUNK, D), jnp.float32),
            pltpu.VMEM((GCHUNK, D), jnp.float32),
            pltpu.VMEM((MCH, ECHUNK), jnp.int32),
            pltpu.VMEM((MCH * ECHUNK,), jnp.float32),
            pltpu.SemaphoreType.DMA,
            pltpu.SemaphoreType.DMA,
        ],
    )(_sc_gather_deg_body)
    return k(table, idx3, dst3, ea3)


def _sc_mp_body(hl2_hbm, src3_hbm, dst3_hbm, ea3_hbm, out_hbm,
                acc_sh, sidx2d, didx2d, ea_v, rows_a, rows_b,
                sem_ga, sem_gb, sem_sa, sem_sb):
    c = lax.axis_index("c")
    s = lax.axis_index("s")
    wid = s * NC + c

    def zr(r, _):
        for j in range(D // L):
            rows_a[r, pl.ds(j * L, L)] = jnp.zeros((L,), jnp.float32)
        return 0
    lax.fori_loop(0, ECHUNK, zr, 0)
    for k in range(ROWS_PER_TILE // ECHUNK):
        sl = pl.ds(s * ROWS_PER_TILE + k * ECHUNK, ECHUNK)
        pltpu.async_copy(rows_a, acc_sh.at[sl], sem_sa)
    for k in range(ROWS_PER_TILE // ECHUNK):
        sl = pl.ds(s * ROWS_PER_TILE + k * ECHUNK, ECHUNK)
        pltpu.make_async_copy(rows_a, acc_sh.at[sl], sem_sa).wait()
    plsc.subcore_barrier()

    rows = (rows_a, rows_b)
    sem_g = (sem_ga, sem_gb)
    sem_s = (sem_sa, sem_sb)

    for phase in range(CH // MCH):
        pltpu.async_copy(src3_hbm.at[wid * NPH + phase], sidx2d, sem_ga)
        pltpu.async_copy(dst3_hbm.at[wid * NPH + phase], didx2d, sem_gb)
        pltpu.async_copy(ea3_hbm.at[wid * NPH + phase], ea_v, sem_sa)
        pltpu.make_async_copy(
            src3_hbm.at[wid * NPH + phase], sidx2d, sem_ga).wait()
        pltpu.make_async_copy(
            dst3_hbm.at[wid * NPH + phase], didx2d, sem_gb).wait()
        pltpu.make_async_copy(
            ea3_hbm.at[wid * NPH + phase], ea_v, sem_sa).wait()
        pltpu.async_copy(hl2_hbm.at[sidx2d.at[0]], rows_a, sem_ga)

        def pair_body(k2, _):
            for u in (0, 1):
                ch = 2 * k2 + u
                rb, rnb = rows[u], rows[1 - u]
                sgb, sgn = sem_g[u], sem_g[1 - u]
                ssb, ssn = sem_s[u], sem_s[1 - u]

                @pl.when(ch > 0)
                def _():
                    pltpu.make_async_copy(
                        rnb, acc_sh.at[didx2d.at[ch - 1]], ssn).wait()

                @pl.when(ch < MCH - 1)
                def _():
                    pltpu.async_copy(hl2_hbm.at[sidx2d.at[ch + 1]], rnb, sgn)

                pltpu.make_async_copy(hl2_hbm.at[sidx2d.at[ch]], rb, sgb).wait()

                @plsc.parallel_loop(0, ECHUNK // L, unroll=2)
                def _(g):
                    ea16 = ea_v[pl.ds(ch * ECHUNK + g * L, L)]
                    for e in range(L):
                        eav = ea16.at[jnp.full((L,), e, jnp.int32)].get(
                            mode="promise_in_bounds")
                        r = g * L + e
                        for j in range(D // L):
                            sl = pl.ds(j * L, L)
                            rb[r, sl] = rb[r, sl] * eav
                pltpu.async_copy(rb, acc_sh.at[didx2d.at[ch]], ssb, add=True)
            return 0
        lax.fori_loop(0, MCH // 2, pair_body, 0)
        pltpu.make_async_copy(
            rows_b, acc_sh.at[didx2d.at[MCH - 1]], sem_sb).wait()
    plsc.subcore_barrier()

    nrb = ROWS_PER_TILE // ECHUNK
    for k in range(nrb):
        b = rows[k % 2]
        sl = pl.ds(s * ROWS_PER_TILE + k * ECHUNK, ECHUNK)
        if k >= 2:
            psl = pl.ds(s * ROWS_PER_TILE + (k - 2) * ECHUNK, ECHUNK)
            pltpu.make_async_copy(b, out_hbm.at[c, psl], sem_s[k % 2]).wait()
        pltpu.async_copy(acc_sh.at[sl], b, sem_g[k % 2])
        pltpu.make_async_copy(acc_sh.at[sl], b, sem_g[k % 2]).wait()
        pltpu.async_copy(b, out_hbm.at[c, sl], sem_s[k % 2])
    for k in range(nrb - 2, nrb):
        b = rows[k % 2]
        sl = pl.ds(s * ROWS_PER_TILE + k * ECHUNK, ECHUNK)
        pltpu.make_async_copy(b, out_hbm.at[c, sl], sem_s[k % 2]).wait()


def _sc_mp(hl2, src3, dst3, ea3):
    mesh = plsc.VectorSubcoreMesh(core_axis_name="c", subcore_axis_name="s")
    k = functools.partial(
        pl.kernel,
        mesh=mesh,
        out_type=jax.ShapeDtypeStruct((NC, N_PAD, D), jnp.float32),
        scratch_types=[
            pltpu.VMEM_SHARED((N_PAD, D), jnp.float32),
            pltpu.VMEM((MCH, ECHUNK), jnp.int32),
            pltpu.VMEM((MCH, ECHUNK), jnp.int32),
            pltpu.VMEM((MCH * ECHUNK,), jnp.float32),
            pltpu.VMEM((ECHUNK, D), jnp.float32),
            pltpu.VMEM((ECHUNK, D), jnp.float32),
            pltpu.SemaphoreType.DMA,
            pltpu.SemaphoreType.DMA,
            pltpu.SemaphoreType.DMA,
            pltpu.SemaphoreType.DMA,
        ],
    )(_sc_mp_body)
    return k(hl2, src3, dst3, ea3)


def _tc_stats_body(h_ref, degp_ref, stats_ref, dis_ref):
    i = pl.program_id(0)
    deg = degp_ref[0] + degp_ref[1]
    dis_ref[...] = jnp.where(deg > 0, lax.rsqrt(jnp.maximum(deg, 1e-12)), 0.0)
    rows = lax.broadcasted_iota(jnp.int32, (BLK, 1), 0) + i * BLK
    m = (rows < N).astype(jnp.float32)
    hm = h_ref[...] * m

    @pl.when(i == 0)
    def _():
        stats_ref[...] = jnp.zeros((8, D), jnp.float32)

    stats_ref[0:1, :] += jnp.sum(hm, axis=0, keepdims=True)
    stats_ref[1:2, :] += jnp.sum(hm * h_ref[...], axis=0, keepdims=True)


def _tc_stats(h, degp3):
    return pl.pallas_call(
        _tc_stats_body,
        grid=(NBLK,),
        in_specs=[
            pl.BlockSpec((BLK, D), lambda i: (i, 0)),
            pl.BlockSpec((NC, BLK, 1), lambda i: (0, i, 0)),
        ],
        out_specs=[
            pl.BlockSpec((8, D), lambda i: (0, 0)),
            pl.BlockSpec((BLK, 1), lambda i: (i, 0)),
        ],
        out_shape=[
            jax.ShapeDtypeStruct((8, D), jnp.float32),
            jax.ShapeDtypeStruct((N_PAD, 1), jnp.float32),
        ],
    )(h, degp3)


def _tc_mm_body(h_ref, stats_ref, g_ref, be_ref, w_ref, dis_ref, out_ref):
    mean = stats_ref[0:1, :] * (1.0 / N)
    var = stats_ref[1:2, :] * (1.0 / N) - mean * mean
    alpha = g_ref[...] * lax.rsqrt(var + 1e-5)
    beta = be_ref[...] - mean * alpha
    hb = h_ref[...] * alpha + beta
    hl = lax.dot_general(hb, w_ref[...], (((1,), (1,)), ((), ())),
                         preferred_element_type=jnp.float32)
    out_ref[...] = hl * dis_ref[...]


def _tc_mm(h, stats, g2, be2, W, dis):
    return pl.pallas_call(
        _tc_mm_body,
        grid=(NBLK,),
        in_specs=[
            pl.BlockSpec((BLK, D), lambda i: (i, 0)),
            pl.BlockSpec((8, D), lambda i: (0, 0)),
            pl.BlockSpec((1, D), lambda i: (0, 0)),
            pl.BlockSpec((1, D), lambda i: (0, 0)),
            pl.BlockSpec((D, D), lambda i: (0, 0)),
            pl.BlockSpec((BLK, 1), lambda i: (i, 0)),
        ],
        out_specs=pl.BlockSpec((BLK, D), lambda i: (i, 0)),
        out_shape=jax.ShapeDtypeStruct((N_PAD, D), jnp.float32),
    )(h, stats, g2, be2, W, dis)


def _tc_ep_body(p_ref, dis_ref, b_ref, h_ref, stats_ref):
    i = pl.program_id(0)
    acc = p_ref[0] + p_ref[1]
    h = jax.nn.relu(acc * dis_ref[...] + b_ref[...])
    h_ref[...] = h
    rows = lax.broadcasted_iota(jnp.int32, (BLK, 1), 0) + i * BLK
    m = (rows < N).astype(jnp.float32)
    hm = h * m

    @pl.when(i == 0)
    def _():
        stats_ref[...] = jnp.zeros((8, D), jnp.float32)

    stats_ref[0:1, :] += jnp.sum(hm, axis=0, keepdims=True)
    stats_ref[1:2, :] += jnp.sum(hm * h, axis=0, keepdims=True)


def _tc_ep(P, dis, b2):
    return pl.pallas_call(
        _tc_ep_body,
        grid=(NBLK,),
        in_specs=[
            pl.BlockSpec((NC, BLK, D), lambda i: (0, i, 0)),
            pl.BlockSpec((BLK, 1), lambda i: (i, 0)),
            pl.BlockSpec((1, D), lambda i: (0, 0)),
        ],
        out_specs=[
            pl.BlockSpec((BLK, D), lambda i: (i, 0)),
            pl.BlockSpec((8, D), lambda i: (0, 0)),
        ],
        out_shape=[
            jax.ShapeDtypeStruct((N_PAD, D), jnp.float32),
            jax.ShapeDtypeStruct((8, D), jnp.float32),
        ],
    )(P, dis, b2)


def _tc_ep_pool_body(p_ref, dis_ref, b_ref, tf_ref, batch_ref,
                     out_ref, pool_acc, s_acc):
    i = pl.program_id(0)
    acc = p_ref[0] + p_ref[1]
    h2 = jax.nn.relu(acc * dis_ref[...] + b_ref[...])
    ex = jnp.exp(tf_ref[...])
    ids = lax.broadcasted_iota(jnp.int32, (G, 1), 0)
    onehot = (batch_ref[0] == ids).astype(jnp.float32)

    @pl.when(i == 0)
    def _():
        pool_acc[...] = jnp.zeros((G, D), jnp.float32)
        s_acc[...] = jnp.zeros((G, 1), jnp.float32)

    hw = h2 * ex
    pool_acc[...] += lax.dot_general(onehot, hw, (((1,), (0,)), ((), ())),
                                     preferred_element_type=jnp.float32)
    s_acc[...] += lax.dot_general(onehot, ex, (((1,), (0,)), ((), ())),
                                  preferred_element_type=jnp.float32)

    @pl.when(i == NBLK - 1)
    def _():
        out_ref[...] = pool_acc[...] / (s_acc[...] + 1e-16)


def _tc_ep_pool(P, dis, b2, tf_col, batch3):
    return pl.pallas_call(
        _tc_ep_pool_body,
        grid=(NBLK,),
        in_specs=[
            pl.BlockSpec((NC, BLK, D), lambda i: (0, i, 0)),
            pl.BlockSpec((BLK, 1), lambda i: (i, 0)),
            pl.BlockSpec((1, D), lambda i: (0, 0)),
            pl.BlockSpec((BLK, 1), lambda i: (i, 0)),
            pl.BlockSpec((1, 1, BLK), lambda i: (i, 0, 0)),
        ],
        out_specs=pl.BlockSpec((G, D), lambda i: (0, 0)),
        out_shape=jax.ShapeDtypeStruct((G, D), jnp.float32),
        scratch_shapes=[
            pltpu.VMEM((G, D), jnp.float32),
            pltpu.VMEM((G, 1), jnp.float32),
        ],
    )(P, dis, b2, tf_col, batch3)


def kernel(x, edge_index, batch, edge_attr, emb_table,
           bn_gamma0, bn_beta0, W0, b0,
           bn_gamma1, bn_beta1, W1, b1):
    x_idx = x[:, 0].astype(jnp.int32)
    tfidf = x[:, 1]
    idx_pad = jnp.concatenate(
        [x_idx, jnp.arange(N_PAD - N, dtype=jnp.int32)])
    idx3 = idx_pad.reshape(NW, EMB_CH, GCHUNK)

    src = edge_index[0].astype(jnp.int32)
    dst = edge_index[1].astype(jnp.int32)
    epad = E_PAD - E
    fill = jnp.arange(epad, dtype=jnp.int32) % N
    src3 = jnp.concatenate([src, fill]).reshape(NW * NPH, MCH, ECHUNK)
    dst3 = jnp.concatenate([dst, fill]).reshape(NW * NPH, MCH, ECHUNK)
    ea3 = jnp.concatenate(
        [edge_attr, jnp.zeros((epad,), jnp.float32)]).reshape(
            NW * NPH, MCH * ECHUNK)

    h0, degp = _sc_gather_deg(emb_table, idx3, dst3, ea3)
    degp3 = degp.reshape(NC, N_PAD, 1)
    stats0, dis = _tc_stats(h0, degp3)
    hl2_0 = _tc_mm(h0, stats0, bn_gamma0.reshape(1, D),
                   bn_beta0.reshape(1, D), W0, dis)
    P0 = _sc_mp(hl2_0, src3, dst3, ea3)
    h1, stats1 = _tc_ep(P0, dis, b0.reshape(1, D))
    hl2_1 = _tc_mm(h1, stats1, bn_gamma1.reshape(1, D),
                   bn_beta1.reshape(1, D), W1, dis)
    P1 = _sc_mp(hl2_1, src3, dst3, ea3)

    tf_col = jnp.concatenate([tfidf, jnp.zeros((N_PAD - N,), jnp.float32)])
    tf_col = tf_col.reshape(N_PAD, 1)
    batch3 = jnp.concatenate([batch.astype(jnp.int32),
                              jnp.full((N_PAD - N,), G, jnp.int32)])
    batch3 = batch3.reshape(NBLK, 1, BLK)

    return _tc_ep_pool(P1, dis, b1.reshape(1, D), tf_col, batch3)
